# baseline (speedup 1.0000x reference)
import functools

import jax
import jax.numpy as jnp
from jax import lax
from jax.experimental import pallas as pl
from jax.experimental.pallas import tpu as pltpu
from jax.experimental.pallas import tpu_sc as plsc

N = 10000
E = 160000
D = 256
H = 128
NC = 2
NS = 16
EPT = E // NS
K = 80
NCH = EPT // K
RP_PAD = 10016
ROWS_PT = N // NS
ZROWS = 32
NZ = 20
MM_BLK = 1000


def _mm_body(x_ref, w_ref, o0_ref, o1_ref):
    r = jnp.dot(x_ref[...], w_ref[...], preferred_element_type=jnp.float32)
    o0_ref[...] = r[:, :H]
    o1_ref[...] = r[:, H:]


_matmul = pl.pallas_call(
    _mm_body,
    grid=(N // MM_BLK,),
    in_specs=[
        pl.BlockSpec((MM_BLK, D), lambda i: (i, 0)),
        pl.BlockSpec((D, D), lambda i: (0, 0)),
    ],
    out_specs=[
        pl.BlockSpec((MM_BLK, H), lambda i: (i, 0)),
        pl.BlockSpec((MM_BLK, H), lambda i: (i, 0)),
    ],
    out_shape=[
        jax.ShapeDtypeStruct((N, H), jnp.float32),
        jax.ShapeDtypeStruct((N, H), jnp.float32),
    ],
)


def _sc_body(xp0, xp1, rp_hbm, col_hbm, zin_hbm, out_hbm,
             rp_v, col_v, dest_v, rows_v, acc,
             sem_g0, sem_g1, sem_s0, sem_s1, sem_z):
    c = lax.axis_index("c")
    s = lax.axis_index("s")
    base = s * EPT

    cp_rp = pltpu.async_copy(rp_hbm, rp_v, sem_g0)
    cp_col = pltpu.async_copy(col_hbm.at[pl.ds(base, EPT)], col_v, sem_g1)

    zrow0 = s * 624
    zcp0 = pltpu.async_copy(
        zin_hbm.at[pl.ds(zrow0, 624), pl.ds(c * H, H)],
        acc.at[pl.ds(zrow0, 624)], sem_z)
    zt = NS * 624

    @pl.when(s == NS - 1)
    def _():
        pltpu.async_copy(zin_hbm.at[pl.ds(zt, N - zt), pl.ds(c * H, H)],
                         acc.at[pl.ds(zt, N - zt)], sem_z)

    cp_rp.wait()
    cp_col.wait()

    NG = K // 16
    iota16 = lax.iota(jnp.int32, 16)

    def dest_chunk(j, slot):
        poss = [base + j * K + g * 16 + iota16 for g in range(NG)]
        los = [jnp.zeros((16,), jnp.int32) for _ in range(NG)]
        his = [jnp.full((16,), N, jnp.int32) for _ in range(NG)]
        for _ in range(14):
            mids = [(los[g] + his[g] + 1) >> 1 for g in range(NG)]
            vs = [plsc.load_gather(rp_v, [mids[g]]) for g in range(NG)]
            for g in range(NG):
                cond = vs[g] <= poss[g]
                los[g] = jnp.where(cond, mids[g], los[g])
                his[g] = jnp.where(cond, his[g], mids[g] - 1)
        for g in range(NG):
            dest_v[slot, pl.ds(g * 16, 16)] = los[g]

    zcp0.wait()

    @pl.when(s == NS - 1)
    def _():
        pltpu.make_async_copy(
            zin_hbm.at[pl.ds(zt, N - zt), pl.ds(c * H, H)],
            acc.at[pl.ds(zt, N - zt)], sem_z).wait()

    plsc.subcore_barrier()

    def pipeline(xp):
        def g_src(j):
            return xp.at[col_v.at[pl.ds(j * K, K)]]

        gbuf = (rows_v.at[0], rows_v.at[1])
        gsem = (sem_g0, sem_g1)
        ssem = (sem_s0, sem_s1)

        def fire_gather(j, b):
            pltpu.async_copy(g_src(j), gbuf[b], gsem[b])

        def wait_gather(j, b):
            pltpu.make_async_copy(g_src(j), gbuf[b], gsem[b]).wait()

        def fire_scatter(b):
            pltpu.async_copy(gbuf[b], acc.at[dest_v.at[b]], ssem[b],
                             add=True)

        def wait_scatter(b):
            pltpu.make_async_copy(gbuf[b], acc.at[dest_v.at[b]],
                                  ssem[b]).wait()

        dest_chunk(0, 0)
        fire_gather(0, 0)
        fire_gather(1, 1)
        dest_chunk(1, 1)
        wait_gather(0, 0)
        fire_scatter(0)

        def step(j, b):
            bn = 1 - b
            wait_scatter(bn)
            fire_gather(j + 1, bn)
            dest_chunk(j + 1, bn)
            wait_gather(j, b)
            fire_scatter(b)

        def mloop(i, carry):
            step(2 * i + 1, 1)
            step(2 * i + 2, 0)
            return carry
        lax.fori_loop(0, (NCH - 3) // 2, mloop, 0)

        step(NCH - 2, (NCH - 2) % 2)
        jn1 = NCH - 1
        wait_gather(jn1, jn1 % 2)
        fire_scatter(jn1 % 2)
        wait_scatter((NCH - 2) % 2)
        wait_scatter(jn1 % 2)

    @pl.when(c == 0)
    def _():
        pipeline(xp0)

    @pl.when(c == 1)
    def _():
        pipeline(xp1)

    plsc.subcore_barrier()

    r0 = s * 624
    wcp = pltpu.async_copy(acc.at[pl.ds(r0, 624)],
                           out_hbm.at[pl.ds(r0, 624), pl.ds(c * H, H)],
                           sem_z)

    @pl.when(s == NS - 1)
    def _():
        pltpu.async_copy(acc.at[pl.ds(zt, N - zt)],
                         out_hbm.at[pl.ds(zt, N - zt), pl.ds(c * H, H)],
                         sem_z)

    wcp.wait()

    @pl.when(s == NS - 1)
    def _():
        pltpu.make_async_copy(acc.at[pl.ds(zt, N - zt)],
                              out_hbm.at[pl.ds(zt, N - zt),
                                         pl.ds(c * H, H)], sem_z).wait()


_sc_spmm = functools.partial(
    pl.kernel,
    out_type=jax.ShapeDtypeStruct((N, D), jnp.float32),
    mesh=plsc.VectorSubcoreMesh(
        core_axis_name="c", subcore_axis_name="s", num_cores=NC,
        num_subcores=NS),
    scratch_types=[
        pltpu.VMEM((RP_PAD,), jnp.int32),
        pltpu.VMEM((EPT,), jnp.int32),
        pltpu.VMEM((2, K), jnp.int32),
        pltpu.VMEM((2, K, H), jnp.float32),
        pltpu.VMEM_SHARED((N, H), jnp.float32),
        pltpu.SemaphoreType.DMA,
        pltpu.SemaphoreType.DMA,
        pltpu.SemaphoreType.DMA,
        pltpu.SemaphoreType.DMA,
        pltpu.SemaphoreType.DMA,
    ],
    compiler_params=pltpu.CompilerParams(needs_layout_passes=False),
)(_sc_body)


def kernel(X, weights, row_pointers, column_index, blockPartition,
           edgeToColumn, edgeToRow, hybrid_type, row_nzr, col_nzr, output):
    xp0, xp1 = _matmul(X, weights)
    rp_pad = jnp.concatenate(
        [row_pointers.astype(jnp.int32),
         jnp.full((RP_PAD - (N + 1),), E, jnp.int32)])
    return _sc_spmm(xp0, xp1, rp_pad, column_index, output)


# speedup vs baseline: 1.0545x; 1.0545x over previous
import functools

import jax
import jax.numpy as jnp
from jax import lax
from jax.experimental import pallas as pl
from jax.experimental.pallas import tpu as pltpu
from jax.experimental.pallas import tpu_sc as plsc

N = 10000
E = 160000
D = 256
H = 128
NC = 2
NS = 16
EPT = E // NS
K = 128
NCH = -(-EPT // K)
EPT_PAD = NCH * K
COL_PAD = (NS - 1) * EPT + EPT_PAD
RP_PAD = 10016
ROWS_PT = N // NS
ZROWS = 32
NZ = 20
MM_BLK = 1000


def _mm_body(x_ref, w_ref, o0_ref, o1_ref):
    r = jnp.dot(x_ref[...], w_ref[...], preferred_element_type=jnp.float32)
    o0_ref[...] = r[:, :H]
    o1_ref[...] = r[:, H:]


_matmul = pl.pallas_call(
    _mm_body,
    grid=(N // MM_BLK,),
    in_specs=[
        pl.BlockSpec((MM_BLK, D), lambda i: (i, 0)),
        pl.BlockSpec((D, D), lambda i: (0, 0)),
    ],
    out_specs=[
        pl.BlockSpec((MM_BLK, H), lambda i: (i, 0)),
        pl.BlockSpec((MM_BLK, H), lambda i: (i, 0)),
    ],
    out_shape=[
        jax.ShapeDtypeStruct((N, H), jnp.float32),
        jax.ShapeDtypeStruct((N, H), jnp.float32),
    ],
)


def _sc_body(xp0, xp1, rp_hbm, col_hbm, out_hbm,
             rp_v, colr_v, dest_v, rows_v, zbuf, acc,
             sem_g0, sem_g1, sem_s0, sem_s1, sem_c0, sem_c1, sem_z):
    c = lax.axis_index("c")
    s = lax.axis_index("s")
    base = s * EPT

    cp_rp = pltpu.async_copy(rp_hbm, rp_v, sem_g0)

    z16 = jnp.zeros((16,), jnp.float32)

    def zrow(r, carry):
        for f in range(H // 16):
            zbuf[r, pl.ds(f * 16, 16)] = z16
        return carry
    lax.fori_loop(0, ZROWS, zrow, 0)

    def zfire(k, carry):
        r0 = jnp.minimum(s * ROWS_PT + k * ZROWS, N - ZROWS)
        pltpu.async_copy(zbuf, acc.at[pl.ds(r0, ZROWS)], sem_z)
        return carry
    lax.fori_loop(0, NZ, zfire, 0)

    cp_rp.wait()

    NG = K // 16
    iota16 = lax.iota(jnp.int32, 16)

    def dest_chunk(j, slot):
        poss = [base + j * K + g * 16 + iota16 for g in range(NG)]
        los = [jnp.zeros((16,), jnp.int32) for _ in range(NG)]
        his = [jnp.full((16,), N, jnp.int32) for _ in range(NG)]
        for _ in range(14):
            mids = [(los[g] + his[g] + 1) >> 1 for g in range(NG)]
            vs = [plsc.load_gather(rp_v, [mids[g]]) for g in range(NG)]
            for g in range(NG):
                cond = vs[g] <= poss[g]
                los[g] = jnp.where(cond, mids[g], los[g])
                his[g] = jnp.where(cond, his[g], mids[g] - 1)
        for g in range(NG):
            d = jnp.where(poss[g] < base + EPT, los[g], N)
            dest_v[slot, pl.ds(g * 16, 16)] = d

    def zdrain(k, carry):
        pltpu.make_async_copy(zbuf, acc.at[pl.ds(0, ZROWS)], sem_z).wait()
        return carry
    lax.fori_loop(0, NZ, zdrain, 0)

    plsc.subcore_barrier()

    def pipeline(xp):
        gbuf = (rows_v.at[0], rows_v.at[1])
        gsem = (sem_g0, sem_g1)
        ssem = (sem_s0, sem_s1)
        csem = (sem_c0, sem_c1)

        def fire_col(j, slot):
            pltpu.async_copy(col_hbm.at[pl.ds(base + j * K, K)],
                             colr_v.at[slot], csem[slot])

        def wait_col(j, slot):
            pltpu.make_async_copy(col_hbm.at[pl.ds(base + j * K, K)],
                                  colr_v.at[slot], csem[slot]).wait()

        def fire_gather(j, b):
            pltpu.async_copy(xp.at[colr_v.at[b]], gbuf[b], gsem[b])

        def wait_gather(j, b):
            pltpu.make_async_copy(xp.at[colr_v.at[b]], gbuf[b],
                                  gsem[b]).wait()

        def fire_scatter(b):
            pltpu.async_copy(gbuf[b], acc.at[dest_v.at[b]], ssem[b],
                             add=True)

        def wait_scatter(b):
            pltpu.make_async_copy(gbuf[b], acc.at[dest_v.at[b]],
                                  ssem[b]).wait()

        fire_col(0, 0)
        fire_col(1, 1)
        dest_chunk(0, 0)
        dest_chunk(1, 1)
        wait_col(0, 0)
        fire_gather(0, 0)
        wait_col(1, 1)
        fire_gather(1, 1)
        wait_gather(0, 0)
        fire_col(2, 0)
        fire_scatter(0)

        def step(j, b, last=False):
            bn = 1 - b
            wait_scatter(bn)
            wait_col(j + 1, bn)
            fire_gather(j + 1, bn)
            dest_chunk(j + 1, bn)
            wait_gather(j, b)
            if not last:
                fire_col(j + 2, b)
            fire_scatter(b)

        def mloop(i, carry):
            step(2 * i + 1, 1)
            step(2 * i + 2, 0)
            return carry
        lax.fori_loop(0, (NCH - 3) // 2, mloop, 0)

        step(NCH - 2, (NCH - 2) % 2, last=True)
        jn1 = NCH - 1
        wait_gather(jn1, jn1 % 2)
        fire_scatter(jn1 % 2)
        wait_scatter((NCH - 2) % 2)
        wait_scatter(jn1 % 2)

    @pl.when(c == 0)
    def _():
        pipeline(xp0)

    @pl.when(c == 1)
    def _():
        pipeline(xp1)

    plsc.subcore_barrier()

    r0 = s * 624
    pltpu.sync_copy(acc.at[pl.ds(r0, 624)],
                    out_hbm.at[pl.ds(r0, 624), pl.ds(c * H, H)])

    @pl.when(s == NS - 1)
    def _():
        pltpu.sync_copy(acc.at[pl.ds(NS * 624, N - NS * 624)],
                        out_hbm.at[pl.ds(NS * 624, N - NS * 624),
                                   pl.ds(c * H, H)])


_sc_spmm = functools.partial(
    pl.kernel,
    out_type=jax.ShapeDtypeStruct((N, D), jnp.float32),
    mesh=plsc.VectorSubcoreMesh(
        core_axis_name="c", subcore_axis_name="s", num_cores=NC,
        num_subcores=NS),
    scratch_types=[
        pltpu.VMEM((RP_PAD,), jnp.int32),
        pltpu.VMEM((2, K), jnp.int32),
        pltpu.VMEM((2, K), jnp.int32),
        pltpu.VMEM((2, K, H), jnp.float32),
        pltpu.VMEM((ZROWS, H), jnp.float32),
        pltpu.VMEM_SHARED((N + 8, H), jnp.float32),
        pltpu.SemaphoreType.DMA,
        pltpu.SemaphoreType.DMA,
        pltpu.SemaphoreType.DMA,
        pltpu.SemaphoreType.DMA,
        pltpu.SemaphoreType.DMA,
        pltpu.SemaphoreType.DMA,
        pltpu.SemaphoreType.DMA,
    ],
    compiler_params=pltpu.CompilerParams(needs_layout_passes=False),
)(_sc_body)


def kernel(X, weights, row_pointers, column_index, blockPartition,
           edgeToColumn, edgeToRow, hybrid_type, row_nzr, col_nzr, output):
    xp0, xp1 = _matmul(X, weights)
    rp_pad = jnp.concatenate(
        [row_pointers.astype(jnp.int32),
         jnp.full((RP_PAD - (N + 1),), E, jnp.int32)])
    col_pad = jnp.concatenate(
        [column_index.astype(jnp.int32),
         jnp.zeros((COL_PAD - E,), jnp.int32)])
    return _sc_spmm(xp0, xp1, rp_pad, col_pad)


# speedup vs baseline: 1.0732x; 1.0178x over previous
import functools

import jax
import jax.numpy as jnp
from jax import lax
from jax.experimental import pallas as pl
from jax.experimental.pallas import tpu as pltpu
from jax.experimental.pallas import tpu_sc as plsc

N = 10000
E = 160000
D = 256
H = 128
NC = 2
NS = 16
EPT = E // NS
K = 128
NCH = -(-EPT // K)
EPT_PAD = NCH * K
COL_PAD = (NS - 1) * EPT + EPT_PAD
RP_PAD = 10016
ROWS_PT = N // NS
ZROWS = 32
NZ = 20
MM_BLK = 1000


def _mm_body(x_ref, w_ref, o0_ref, o1_ref):
    r = jnp.dot(x_ref[...], w_ref[...], preferred_element_type=jnp.float32)
    o0_ref[...] = r[:, :H]
    o1_ref[...] = r[:, H:]


_matmul = pl.pallas_call(
    _mm_body,
    grid=(N // MM_BLK,),
    in_specs=[
        pl.BlockSpec((MM_BLK, D), lambda i: (i, 0)),
        pl.BlockSpec((D, D), lambda i: (0, 0)),
    ],
    out_specs=[
        pl.BlockSpec((MM_BLK, H), lambda i: (i, 0)),
        pl.BlockSpec((MM_BLK, H), lambda i: (i, 0)),
    ],
    out_shape=[
        jax.ShapeDtypeStruct((N, H), jnp.float32),
        jax.ShapeDtypeStruct((N, H), jnp.float32),
    ],
)


def _sc_body(xp0, xp1, rp_hbm, col_hbm, out_hbm,
             rp_v, colr_v, dest_v, rows_v, zbuf, acc,
             sem_g0, sem_g1, sem_s0, sem_s1, sem_c0, sem_c1, sem_z):
    c = lax.axis_index("c")
    s = lax.axis_index("s")
    base = s * EPT

    cp_rp = pltpu.async_copy(rp_hbm, rp_v.at[pl.ds(0, N + 1)], sem_g0)

    z16 = jnp.zeros((16,), jnp.float32)

    def zrow(r, carry):
        for f in range(H // 16):
            zbuf[r, pl.ds(f * 16, 16)] = z16
        return carry
    lax.fori_loop(0, ZROWS, zrow, 0)

    def zfire(k, carry):
        r0 = jnp.minimum(s * ROWS_PT + k * ZROWS, N - ZROWS)
        pltpu.async_copy(zbuf, acc.at[pl.ds(r0, ZROWS)], sem_z)
        return carry
    lax.fori_loop(0, NZ, zfire, 0)

    cp_rp.wait()

    NG = K // 16
    iota16 = lax.iota(jnp.int32, 16)

    def dest_chunk(j, slot):
        poss = [base + j * K + g * 16 + iota16 for g in range(NG)]
        los = [jnp.zeros((16,), jnp.int32) for _ in range(NG)]
        his = [jnp.full((16,), N, jnp.int32) for _ in range(NG)]
        for _ in range(14):
            mids = [(los[g] + his[g] + 1) >> 1 for g in range(NG)]
            vs = [plsc.load_gather(rp_v, [mids[g]]) for g in range(NG)]
            for g in range(NG):
                cond = vs[g] <= poss[g]
                los[g] = jnp.where(cond, mids[g], los[g])
                his[g] = jnp.where(cond, his[g], mids[g] - 1)
        for g in range(NG):
            d = jnp.where(poss[g] < base + EPT, los[g], N)
            dest_v[slot, pl.ds(g * 16, 16)] = d

    def zdrain(k, carry):
        pltpu.make_async_copy(zbuf, acc.at[pl.ds(0, ZROWS)], sem_z).wait()
        return carry
    lax.fori_loop(0, NZ, zdrain, 0)

    plsc.subcore_barrier()

    def pipeline(xp):
        gbuf = (rows_v.at[0], rows_v.at[1])
        gsem = (sem_g0, sem_g1)
        ssem = (sem_s0, sem_s1)
        csem = (sem_c0, sem_c1)

        TAILR = E - (NS - 1) * EPT - (NCH - 1) * K

        def fire_col(j, slot, tail=False):
            if tail:
                @pl.when(s < NS - 1)
                def _():
                    pltpu.async_copy(col_hbm.at[pl.ds(base + j * K, K)],
                                     colr_v.at[slot], csem[slot])

                @pl.when(s == NS - 1)
                def _():
                    pltpu.async_copy(
                        col_hbm.at[pl.ds(base + j * K, TAILR)],
                        colr_v.at[slot, pl.ds(0, TAILR)], csem[slot])
            else:
                pltpu.async_copy(col_hbm.at[pl.ds(base + j * K, K)],
                                 colr_v.at[slot], csem[slot])

        def wait_col(j, slot, tail=False):
            if tail:
                @pl.when(s < NS - 1)
                def _():
                    pltpu.make_async_copy(
                        col_hbm.at[pl.ds(base + j * K, K)],
                        colr_v.at[slot], csem[slot]).wait()

                @pl.when(s == NS - 1)
                def _():
                    pltpu.make_async_copy(
                        col_hbm.at[pl.ds(base + j * K, TAILR)],
                        colr_v.at[slot, pl.ds(0, TAILR)], csem[slot]).wait()
            else:
                pltpu.make_async_copy(col_hbm.at[pl.ds(base + j * K, K)],
                                      colr_v.at[slot], csem[slot]).wait()

        def fire_gather(j, b):
            pltpu.async_copy(xp.at[colr_v.at[b]], gbuf[b], gsem[b])

        def wait_gather(j, b):
            pltpu.make_async_copy(xp.at[colr_v.at[b]], gbuf[b],
                                  gsem[b]).wait()

        def fire_scatter(b):
            pltpu.async_copy(gbuf[b], acc.at[dest_v.at[b]], ssem[b],
                             add=True)

        def wait_scatter(b):
            pltpu.make_async_copy(gbuf[b], acc.at[dest_v.at[b]],
                                  ssem[b]).wait()

        fire_col(0, 0)
        fire_col(1, 1)
        dest_chunk(0, 0)
        dest_chunk(1, 1)
        wait_col(0, 0)
        fire_gather(0, 0)
        wait_col(1, 1)
        fire_gather(1, 1)
        wait_gather(0, 0)
        fire_col(2, 0)
        fire_scatter(0)

        def step(j, b, last=False, wtail=False, ftail=False):
            bn = 1 - b
            wait_scatter(bn)
            wait_col(j + 1, bn, tail=wtail)
            fire_gather(j + 1, bn)
            dest_chunk(j + 1, bn)
            wait_gather(j, b)
            if not last:
                fire_col(j + 2, b, tail=ftail)
            fire_scatter(b)

        def mloop(i, carry):
            step(2 * i + 1, 1)
            step(2 * i + 2, 0)
            return carry
        lax.fori_loop(0, (NCH - 5) // 2, mloop, 0)

        step(NCH - 4, (NCH - 4) % 2)
        step(NCH - 3, (NCH - 3) % 2, ftail=True)
        step(NCH - 2, (NCH - 2) % 2, last=True, wtail=True)
        jn1 = NCH - 1
        wait_gather(jn1, jn1 % 2)
        fire_scatter(jn1 % 2)
        wait_scatter((NCH - 2) % 2)
        wait_scatter(jn1 % 2)

    @pl.when(c == 0)
    def _():
        pipeline(xp0)

    @pl.when(c == 1)
    def _():
        pipeline(xp1)

    plsc.subcore_barrier()

    r0 = s * 624
    pltpu.sync_copy(acc.at[pl.ds(r0, 624)],
                    out_hbm.at[pl.ds(r0, 624), pl.ds(c * H, H)])

    @pl.when(s == NS - 1)
    def _():
        pltpu.sync_copy(acc.at[pl.ds(NS * 624, N - NS * 624)],
                        out_hbm.at[pl.ds(NS * 624, N - NS * 624),
                                   pl.ds(c * H, H)])


_sc_spmm = functools.partial(
    pl.kernel,
    out_type=jax.ShapeDtypeStruct((N, D), jnp.float32),
    mesh=plsc.VectorSubcoreMesh(
        core_axis_name="c", subcore_axis_name="s", num_cores=NC,
        num_subcores=NS),
    scratch_types=[
        pltpu.VMEM((RP_PAD,), jnp.int32),
        pltpu.VMEM((2, K), jnp.int32),
        pltpu.VMEM((2, K), jnp.int32),
        pltpu.VMEM((2, K, H), jnp.float32),
        pltpu.VMEM((ZROWS, H), jnp.float32),
        pltpu.VMEM_SHARED((N + 8, H), jnp.float32),
        pltpu.SemaphoreType.DMA,
        pltpu.SemaphoreType.DMA,
        pltpu.SemaphoreType.DMA,
        pltpu.SemaphoreType.DMA,
        pltpu.SemaphoreType.DMA,
        pltpu.SemaphoreType.DMA,
        pltpu.SemaphoreType.DMA,
    ],
    compiler_params=pltpu.CompilerParams(needs_layout_passes=False),
)(_sc_body)


def kernel(X, weights, row_pointers, column_index, blockPartition,
           edgeToColumn, edgeToRow, hybrid_type, row_nzr, col_nzr, output):
    xp0, xp1 = _matmul(X, weights)
    return _sc_spmm(xp0, xp1, row_pointers, column_index)


# speedup vs baseline: 1.0760x; 1.0026x over previous
import functools

import jax
import jax.numpy as jnp
from jax import lax
from jax.experimental import pallas as pl
from jax.experimental.pallas import tpu as pltpu
from jax.experimental.pallas import tpu_sc as plsc

N = 10000
E = 160000
D = 256
H = 128
NC = 2
NS = 16
EPT = E // NS
K = 128
NCH = -(-EPT // K)
RP_PAD = 10016
ROWS_PT = N // NS
ZROWS = 32
NZ = 20
MM_BLK = 1000


def _mm_body(x_ref, w_ref, o0_ref, o1_ref):
    r = jnp.dot(x_ref[...], w_ref[...], preferred_element_type=jnp.float32)
    o0_ref[...] = r[:, :H]
    o1_ref[...] = r[:, H:]


_matmul = pl.pallas_call(
    _mm_body,
    grid=(N // MM_BLK,),
    in_specs=[
        pl.BlockSpec((MM_BLK, D), lambda i: (i, 0)),
        pl.BlockSpec((D, D), lambda i: (0, 0)),
    ],
    out_specs=[
        pl.BlockSpec((MM_BLK, H), lambda i: (i, 0)),
        pl.BlockSpec((MM_BLK, H), lambda i: (i, 0)),
    ],
    out_shape=[
        jax.ShapeDtypeStruct((N, H), jnp.float32),
        jax.ShapeDtypeStruct((N, H), jnp.float32),
    ],
)


def _sc_body(xp0, xp1, rp_hbm, col_hbm, out_hbm,
             rp_v, colr_v, dest_v, rows_v, zbuf, acc,
             sem_g0, sem_g1, sem_s0, sem_s1, sem_c0, sem_c1, sem_z):
    c = lax.axis_index("c")
    s = lax.axis_index("s")
    base = s * EPT

    cp_rp = pltpu.async_copy(rp_hbm, rp_v.at[pl.ds(0, N + 1)], sem_g0)

    z16 = jnp.zeros((16,), jnp.float32)

    def zrow(r, carry):
        for f in range(H // 16):
            zbuf[r, pl.ds(f * 16, 16)] = z16
        return carry
    lax.fori_loop(0, ZROWS, zrow, 0)

    def zfire(k, carry):
        r0 = jnp.minimum(s * ROWS_PT + k * ZROWS, N - ZROWS)
        pltpu.async_copy(zbuf, acc.at[pl.ds(r0, ZROWS)], sem_z)
        return carry
    lax.fori_loop(0, NZ, zfire, 0)

    cp_rp.wait()

    NG = K // 16
    iota16 = lax.iota(jnp.int32, 16)

    def dest_chunk(j, slot):
        poss = [base + j * K + g * 16 + iota16 for g in range(NG)]
        los = [jnp.zeros((16,), jnp.int32) for _ in range(NG)]
        his = [jnp.full((16,), N, jnp.int32) for _ in range(NG)]
        for _ in range(14):
            mids = [(los[g] + his[g] + 1) >> 1 for g in range(NG)]
            vs = [plsc.load_gather(rp_v, [mids[g]]) for g in range(NG)]
            for g in range(NG):
                cond = vs[g] <= poss[g]
                los[g] = jnp.where(cond, mids[g], los[g])
                his[g] = jnp.where(cond, his[g], mids[g] - 1)
        for g in range(NG):
            d = jnp.where(poss[g] < base + EPT, los[g], N)
            dest_v[slot, pl.ds(g * 16, 16)] = d

    def zdrain(k, carry):
        pltpu.make_async_copy(zbuf, acc.at[pl.ds(0, ZROWS)], sem_z).wait()
        return carry
    lax.fori_loop(0, NZ, zdrain, 0)

    plsc.subcore_barrier()

    def pipeline(xp):
        gbuf = (rows_v.at[0], rows_v.at[1])
        gsem = (sem_g0, sem_g1)
        ssem = (sem_s0, sem_s1)
        csem = (sem_c0, sem_c1)

        TAILR = E - (NS - 1) * EPT - (NCH - 1) * K

        def fire_col(j, slot, tail=False):
            if tail:
                @pl.when(s < NS - 1)
                def _():
                    pltpu.async_copy(col_hbm.at[pl.ds(base + j * K, K)],
                                     colr_v.at[slot], csem[slot])

                @pl.when(s == NS - 1)
                def _():
                    pltpu.async_copy(
                        col_hbm.at[pl.ds(base + j * K, TAILR)],
                        colr_v.at[slot, pl.ds(0, TAILR)], csem[slot])
            else:
                pltpu.async_copy(col_hbm.at[pl.ds(base + j * K, K)],
                                 colr_v.at[slot], csem[slot])

        def wait_col(j, slot, tail=False):
            if tail:
                @pl.when(s < NS - 1)
                def _():
                    pltpu.make_async_copy(
                        col_hbm.at[pl.ds(base + j * K, K)],
                        colr_v.at[slot], csem[slot]).wait()

                @pl.when(s == NS - 1)
                def _():
                    pltpu.make_async_copy(
                        col_hbm.at[pl.ds(base + j * K, TAILR)],
                        colr_v.at[slot, pl.ds(0, TAILR)], csem[slot]).wait()
            else:
                pltpu.make_async_copy(col_hbm.at[pl.ds(base + j * K, K)],
                                      colr_v.at[slot], csem[slot]).wait()

        def fire_gather(j, b):
            pltpu.async_copy(xp.at[colr_v.at[b]], gbuf[b], gsem[b])

        def wait_gather(j, b):
            pltpu.make_async_copy(xp.at[colr_v.at[b]], gbuf[b],
                                  gsem[b]).wait()

        def fire_scatter(b):
            pltpu.async_copy(gbuf[b], acc.at[dest_v.at[b]], ssem[b],
                             add=True)

        def wait_scatter(b):
            pltpu.make_async_copy(gbuf[b], acc.at[dest_v.at[b]],
                                  ssem[b]).wait()

        fire_col(0, 0)
        fire_col(1, 1)
        dest_chunk(0, 0)
        dest_chunk(1, 1)
        wait_col(0, 0)
        fire_gather(0, 0)
        wait_col(1, 1)
        fire_gather(1, 1)
        wait_gather(0, 0)
        fire_col(2, 0)
        fire_scatter(0)

        def step(j, b, last=False, wtail=False, ftail=False):
            bn = 1 - b
            wait_scatter(bn)
            wait_col(j + 1, bn, tail=wtail)
            fire_gather(j + 1, bn)
            dest_chunk(j + 1, bn)
            wait_gather(j, b)
            if not last:
                fire_col(j + 2, b, tail=ftail)
            fire_scatter(b)

        def mloop(i, carry):
            step(2 * i + 1, 1)
            step(2 * i + 2, 0)
            return carry
        lax.fori_loop(0, (NCH - 5) // 2, mloop, 0)

        step(NCH - 4, (NCH - 4) % 2)
        step(NCH - 3, (NCH - 3) % 2, ftail=True)
        step(NCH - 2, (NCH - 2) % 2, last=True, wtail=True)
        jn1 = NCH - 1
        wait_gather(jn1, jn1 % 2)
        fire_scatter(jn1 % 2)
        wait_scatter((NCH - 2) % 2)
        wait_scatter(jn1 % 2)

    @pl.when(c == 0)
    def _():
        pipeline(xp0)

    @pl.when(c == 1)
    def _():
        pipeline(xp1)

    plsc.subcore_barrier()

    r0 = s * 624
    pltpu.sync_copy(acc.at[pl.ds(r0, 624)],
                    out_hbm.at[pl.ds(r0, 624), pl.ds(c * H, H)])

    @pl.when(s == NS - 1)
    def _():
        pltpu.sync_copy(acc.at[pl.ds(NS * 624, N - NS * 624)],
                        out_hbm.at[pl.ds(NS * 624, N - NS * 624),
                                   pl.ds(c * H, H)])


_sc_spmm = functools.partial(
    pl.kernel,
    out_type=jax.ShapeDtypeStruct((N, D), jnp.float32),
    mesh=plsc.VectorSubcoreMesh(
        core_axis_name="c", subcore_axis_name="s", num_cores=NC,
        num_subcores=NS),
    scratch_types=[
        pltpu.VMEM((RP_PAD,), jnp.int32),
        pltpu.VMEM((2, K), jnp.int32),
        pltpu.VMEM((2, K), jnp.int32),
        pltpu.VMEM((2, K, H), jnp.float32),
        pltpu.VMEM((ZROWS, H), jnp.float32),
        pltpu.VMEM_SHARED((N + 8, H), jnp.float32),
        pltpu.SemaphoreType.DMA,
        pltpu.SemaphoreType.DMA,
        pltpu.SemaphoreType.DMA,
        pltpu.SemaphoreType.DMA,
        pltpu.SemaphoreType.DMA,
        pltpu.SemaphoreType.DMA,
        pltpu.SemaphoreType.DMA,
    ],
    compiler_params=pltpu.CompilerParams(needs_layout_passes=False),
)(_sc_body)


def kernel(X, weights, row_pointers, column_index, blockPartition,
           edgeToColumn, edgeToRow, hybrid_type, row_nzr, col_nzr, output):
    xp0, xp1 = _matmul(X, weights)
    return _sc_spmm(xp0, xp1, row_pointers, column_index)
